# Initial kernel scaffold; baseline (speedup 1.0000x reference)
import jax
import jax.numpy as jnp
from jax.experimental import pallas as pl


def kernel(atom_features, bond_features, pair_indices, kernel, bias):
    raise NotImplementedError("write your pallas kernel here")


# speedup vs baseline: 1.7170x; 1.7170x over previous
import functools

import jax
import jax.numpy as jnp
from jax import lax
from jax.experimental import pallas as pl
from jax.experimental.pallas import tpu as pltpu
from jax.experimental.pallas import tpu_sc as plsc

ATOM_DIM = 32
BOND_DIM = 16
N_NODES = 10000
N_EDGES = 100000

NW = 32
CHUNK = 128
E_PER_W = 3200
E_PAD = NW * E_PER_W
N_CHUNKS = E_PER_W // CHUNK
N_PAD = 10240
ROWS_PER_TILE = N_PAD // 16


def _gather_body(table_hbm, idx_hbm, out_hbm, idx_small, rows_v, sem):
    cid = lax.axis_index("c")
    sid = lax.axis_index("s")
    wid = sid * 2 + cid
    base = wid * E_PER_W

    def step(j, carry):
        off = j * CHUNK
        pltpu.sync_copy(idx_hbm.at[pl.ds(base + off, CHUNK)], idx_small)
        pltpu.async_copy(table_hbm.at[idx_small],
                         rows_v.at[pl.ds(off, CHUNK)], sem).wait()
        return carry

    lax.fori_loop(0, N_CHUNKS, step, 0)
    pltpu.sync_copy(rows_v, out_hbm.at[pl.ds(base, E_PER_W)])


def _sc_gather(atom_features, src_idx):
    k = functools.partial(
        pl.kernel,
        out_type=jax.ShapeDtypeStruct((E_PAD, ATOM_DIM), jnp.float32),
        mesh=plsc.VectorSubcoreMesh(core_axis_name="c", subcore_axis_name="s"),
        scratch_types=[
            pltpu.VMEM((CHUNK,), jnp.int32),
            pltpu.VMEM((E_PER_W, ATOM_DIM), jnp.float32),
            pltpu.SemaphoreType.DMA,
        ],
        compiler_params=pltpu.CompilerParams(use_tc_tiling_on_sc=False),
    )(_gather_body)
    return k(atom_features, src_idx)


def _scatter_body(msg_hbm, idx_hbm, zeros_hbm, out_hbm,
                  msg_v, idx_small, acc_shared):
    cid = lax.axis_index("c")
    sid = lax.axis_index("s")
    wid = sid * 2 + cid
    base = wid * E_PER_W
    row0 = sid * ROWS_PER_TILE
    pltpu.sync_copy(msg_hbm.at[pl.ds(base, E_PER_W)], msg_v)
    pltpu.sync_copy(zeros_hbm.at[pl.ds(row0, ROWS_PER_TILE)],
                    acc_shared.at[pl.ds(row0, ROWS_PER_TILE)])
    plsc.subcore_barrier()

    def step(j, carry):
        off = j * CHUNK
        pltpu.sync_copy(idx_hbm.at[pl.ds(base + off, CHUNK)], idx_small)
        pltpu.sync_copy(msg_v.at[pl.ds(off, CHUNK)],
                        acc_shared.at[idx_small], add=True)
        return carry

    lax.fori_loop(0, N_CHUNKS, step, 0)
    plsc.subcore_barrier()
    pltpu.sync_copy(acc_shared.at[pl.ds(row0, ROWS_PER_TILE)],
                    out_hbm.at[pl.ds(cid * N_PAD + row0, ROWS_PER_TILE)])


def _sc_scatter(msg, dst_idx, zeros_init):
    k = functools.partial(
        pl.kernel,
        out_type=jax.ShapeDtypeStruct((2 * N_PAD, ATOM_DIM), jnp.float32),
        mesh=plsc.VectorSubcoreMesh(core_axis_name="c", subcore_axis_name="s"),
        scratch_types=[
            pltpu.VMEM((E_PER_W, ATOM_DIM), jnp.float32),
            pltpu.VMEM((CHUNK,), jnp.int32),
            pltpu.VMEM_SHARED((N_PAD, ATOM_DIM), jnp.float32),
        ],
        compiler_params=pltpu.CompilerParams(use_tc_tiling_on_sc=False),
    )(_scatter_body)
    return k(msg, dst_idx, zeros_init)


def _matmul_body(x_ref, bond_ref, w_ref, out_ref):
    y = jnp.dot(x_ref[...], w_ref[...], preferred_element_type=jnp.float32)
    b = bond_ref[...]
    msg = b[:, 16:17] * y[:, 512:544]
    for t in range(BOND_DIM):
        msg = msg + b[:, t:t + 1] * y[:, 32 * t:32 * t + 32]
    out_ref[...] = msg


def _tc_matmul(xg, bond_p, wcat):
    tile = 2048
    grid = (E_PAD // tile,)
    return pl.pallas_call(
        _matmul_body,
        grid=grid,
        in_specs=[
            pl.BlockSpec((tile, ATOM_DIM), lambda i: (i, 0)),
            pl.BlockSpec((tile, 32), lambda i: (i, 0)),
            pl.BlockSpec((ATOM_DIM, 640), lambda i: (0, 0)),
        ],
        out_specs=pl.BlockSpec((tile, ATOM_DIM), lambda i: (i, 0)),
        out_shape=jax.ShapeDtypeStruct((E_PAD, ATOM_DIM), jnp.float32),
    )(xg, bond_p, wcat)


def _combine_body(p_ref, out_ref):
    out_ref[...] = p_ref[0] + p_ref[1]


def _tc_combine(partials):
    return pl.pallas_call(
        _combine_body,
        out_shape=jax.ShapeDtypeStruct((N_PAD, ATOM_DIM), jnp.float32),
    )(partials)


def kernel(atom_features, bond_features, pair_indices, kernel, bias):
    weight = kernel
    src = pair_indices[:, 1].astype(jnp.int32)
    dst = pair_indices[:, 0].astype(jnp.int32)
    src_pad = jnp.concatenate(
        [src, jnp.zeros((E_PAD - N_EDGES,), jnp.int32)])
    dst_pad = jnp.concatenate(
        [dst, jnp.full((E_PAD - N_EDGES,), N_NODES, jnp.int32)])
    bond_p = jnp.zeros((E_PAD, 32), jnp.float32)
    bond_p = bond_p.at[:N_EDGES, :BOND_DIM].set(bond_features)
    bond_p = bond_p.at[:N_EDGES, 16].set(1.0)
    w3 = weight.reshape(BOND_DIM, ATOM_DIM, ATOM_DIM)
    wcat = jnp.transpose(w3, (2, 0, 1)).reshape(ATOM_DIM, BOND_DIM * ATOM_DIM)
    bias_t = bias.reshape(ATOM_DIM, ATOM_DIM).T
    wcat = jnp.concatenate(
        [wcat, bias_t, jnp.zeros((ATOM_DIM, 96), jnp.float32)], axis=1)

    xg = _sc_gather(atom_features, src_pad)
    msg = _tc_matmul(xg, bond_p, wcat)
    zeros_init = jnp.zeros((N_PAD, ATOM_DIM), jnp.float32)
    partials = _sc_scatter(msg, dst_pad, zeros_init)
    out = _tc_combine(partials.reshape(2, N_PAD, ATOM_DIM))
    return out[:N_NODES]


# speedup vs baseline: 4.2334x; 2.4656x over previous
import functools

import jax
import jax.numpy as jnp
from jax import lax
from jax.experimental import pallas as pl
from jax.experimental.pallas import tpu as pltpu
from jax.experimental.pallas import tpu_sc as plsc

ATOM_DIM = 32
BOND_DIM = 16
N_NODES = 10000
N_EDGES = 100000

NW = 32
CHUNK = 128
E_PER_W = 3200
E_PAD = NW * E_PER_W
N_CHUNKS = E_PER_W // CHUNK
N_PAD = 10240
ROWS_PER_TILE = N_PAD // 16


def _gather_body(table_hbm, idx_hbm, out_hbm, idx_v, rows_v, sem):
    cid = lax.axis_index("c")
    sid = lax.axis_index("s")
    wid = sid * 2 + cid
    base = wid * E_PER_W
    pltpu.sync_copy(idx_hbm.at[pl.ds(wid * N_CHUNKS, N_CHUNKS)], idx_v)
    copies = [
        pltpu.async_copy(table_hbm.at[idx_v.at[j]],
                         rows_v.at[pl.ds(j * CHUNK, CHUNK)], sem)
        for j in range(N_CHUNKS)
    ]
    for c in copies:
        c.wait()
    pltpu.sync_copy(rows_v, out_hbm.at[pl.ds(base, E_PER_W)])


def _sc_gather(atom_features, src_idx2d):
    k = functools.partial(
        pl.kernel,
        out_type=jax.ShapeDtypeStruct((E_PAD, ATOM_DIM), jnp.float32),
        mesh=plsc.VectorSubcoreMesh(core_axis_name="c", subcore_axis_name="s"),
        scratch_types=[
            pltpu.VMEM((N_CHUNKS, CHUNK), jnp.int32),
            pltpu.VMEM((E_PER_W, ATOM_DIM), jnp.float32),
            pltpu.SemaphoreType.DMA,
        ],
        compiler_params=pltpu.CompilerParams(use_tc_tiling_on_sc=False),
    )(_gather_body)
    return k(atom_features, src_idx2d)


def _scatter_body(msg_hbm, idx_hbm, zeros_hbm, out_hbm,
                  msg_v, idx_small, acc_shared):
    cid = lax.axis_index("c")
    sid = lax.axis_index("s")
    wid = sid * 2 + cid
    base = wid * E_PER_W
    row0 = sid * ROWS_PER_TILE
    pltpu.sync_copy(msg_hbm.at[pl.ds(base, E_PER_W)], msg_v)
    pltpu.sync_copy(zeros_hbm.at[pl.ds(row0, ROWS_PER_TILE)],
                    acc_shared.at[pl.ds(row0, ROWS_PER_TILE)])
    plsc.subcore_barrier()

    def step(j, carry):
        off = j * CHUNK
        pltpu.sync_copy(idx_hbm.at[pl.ds(base + off, CHUNK)], idx_small)
        pltpu.sync_copy(msg_v.at[pl.ds(off, CHUNK)],
                        acc_shared.at[idx_small], add=True)
        return carry

    lax.fori_loop(0, N_CHUNKS, step, 0)
    plsc.subcore_barrier()
    pltpu.sync_copy(acc_shared.at[pl.ds(row0, ROWS_PER_TILE)],
                    out_hbm.at[pl.ds(cid * N_PAD + row0, ROWS_PER_TILE)])


def _sc_scatter(msg, dst_idx, zeros_init):
    k = functools.partial(
        pl.kernel,
        out_type=jax.ShapeDtypeStruct((2 * N_PAD, ATOM_DIM), jnp.float32),
        mesh=plsc.VectorSubcoreMesh(core_axis_name="c", subcore_axis_name="s"),
        scratch_types=[
            pltpu.VMEM((E_PER_W, ATOM_DIM), jnp.float32),
            pltpu.VMEM((CHUNK,), jnp.int32),
            pltpu.VMEM_SHARED((N_PAD, ATOM_DIM), jnp.float32),
        ],
        compiler_params=pltpu.CompilerParams(use_tc_tiling_on_sc=False),
    )(_scatter_body)
    return k(msg, dst_idx, zeros_init)


def _matmul_body(x_ref, bond_ref, r_ref, s_ref, w4_ref, bt_ref, out_ref):
    x = x_ref[...]
    o = (jnp.dot(bond_ref[...], r_ref[...],
                 preferred_element_type=jnp.float32)
         * jnp.dot(x, s_ref[...], preferred_element_type=jnp.float32))
    msg = (jnp.dot(o, w4_ref[...], preferred_element_type=jnp.float32)
           + jnp.dot(x, bt_ref[...], preferred_element_type=jnp.float32))
    out_ref[...] = msg


def _tc_matmul(xg, bond_pad, rsel, ssel, w4, bias_t):
    tile = 2048
    grid = (E_PAD // tile,)
    return pl.pallas_call(
        _matmul_body,
        grid=grid,
        in_specs=[
            pl.BlockSpec((tile, ATOM_DIM), lambda i: (i, 0)),
            pl.BlockSpec((tile, BOND_DIM), lambda i: (i, 0)),
            pl.BlockSpec((BOND_DIM, 512), lambda i: (0, 0)),
            pl.BlockSpec((ATOM_DIM, 512), lambda i: (0, 0)),
            pl.BlockSpec((512, ATOM_DIM), lambda i: (0, 0)),
            pl.BlockSpec((ATOM_DIM, ATOM_DIM), lambda i: (0, 0)),
        ],
        out_specs=pl.BlockSpec((tile, ATOM_DIM), lambda i: (i, 0)),
        out_shape=jax.ShapeDtypeStruct((E_PAD, ATOM_DIM), jnp.float32),
    )(xg, bond_pad, rsel, ssel, w4, bias_t)


def _combine_body(p_ref, out_ref):
    out_ref[...] = p_ref[0] + p_ref[1]


def _tc_combine(partials):
    return pl.pallas_call(
        _combine_body,
        out_shape=jax.ShapeDtypeStruct((N_PAD, ATOM_DIM), jnp.float32),
    )(partials)


def kernel(atom_features, bond_features, pair_indices, kernel, bias):
    weight = kernel
    src = pair_indices[:, 1].astype(jnp.int32)
    dst = pair_indices[:, 0].astype(jnp.int32)
    src_pad = jnp.concatenate(
        [src, jnp.zeros((E_PAD - N_EDGES,), jnp.int32)])
    dst_pad = jnp.concatenate(
        [dst, jnp.full((E_PAD - N_EDGES,), N_NODES, jnp.int32)])
    bond_pad = jnp.pad(bond_features, ((0, E_PAD - N_EDGES), (0, 0)))
    rsel = jnp.kron(jnp.eye(BOND_DIM, dtype=jnp.float32),
                    jnp.ones((1, ATOM_DIM), jnp.float32))
    ssel = jnp.kron(jnp.ones((1, BOND_DIM), jnp.float32),
                    jnp.eye(ATOM_DIM, dtype=jnp.float32))
    w3 = weight.reshape(BOND_DIM, ATOM_DIM, ATOM_DIM)
    w4 = jnp.transpose(w3, (0, 2, 1)).reshape(BOND_DIM * ATOM_DIM, ATOM_DIM)
    bias_t = bias.reshape(ATOM_DIM, ATOM_DIM).T

    xg = _sc_gather(atom_features, src_pad.reshape(E_PAD // CHUNK, CHUNK))
    msg = _tc_matmul(xg, bond_pad, rsel, ssel, w4, bias_t)
    zeros_init = jnp.zeros((N_PAD, ATOM_DIM), jnp.float32)
    partials = _sc_scatter(msg, dst_pad, zeros_init)
    out = _tc_combine(partials.reshape(2, N_PAD, ATOM_DIM))
    return out[:N_NODES]
